# baseline (speedup 1.0000x reference)
import jax
import jax.numpy as jnp
from jax import lax
from jax.experimental import pallas as pl
from jax.experimental.pallas import tpu as pltpu
from jax.experimental.pallas import tpu_sc as plsc

_K = 25088
_ROW = 224 * 224
_ROWS = 768
_NC, _NS, _L = 2, 16, 16
_NW = _NC * _NS
_RPW = _ROWS // _NW
_NSLICES = _ROW // _L
_ABSM = 0x7FFFFFFF


def _clear_hist(hist_v):
    zeros = jnp.zeros((_L,), jnp.int32)

    def body(j, c):
        hist_v[pl.ds(j * _L, _L)] = zeros
        return c

    lax.fori_loop(0, 256 * _L // _L, body, jnp.int32(0))


def _totals(hist_v, tot_v):
    def body(j, c):
        acc = jnp.zeros((_L,), jnp.int32)
        for l in range(_L):
            acc = acc + hist_v[pl.ds(l * 256 + j * _L, _L)]
        tot_v[pl.ds(j * _L, _L)] = acc
        return c

    lax.fori_loop(0, 256 // _L, body, jnp.int32(0))


def _scan_bucket(tot_v, krem, top):
    def cond(c):
        acc, _ = c
        return acc < krem

    def body(c):
        acc, b = c
        b = b - 1
        return acc + tot_v[pl.ds(b, _L)][0], b

    acc, b = lax.while_loop(cond, body, (jnp.int32(0), jnp.int32(top)))
    krem_next = krem - (acc - tot_v[pl.ds(b, _L)][0])
    return b, krem_next


def _sc_body(x_hbm, thr_hbm, row_v, s2_v, hist_v, tot_v, thr_v):
    wid = lax.axis_index("s") * _NC + lax.axis_index("c")
    iota = lax.iota(jnp.int32, _L)
    lane_base = iota * 256
    ones = jnp.ones((_L,), jnp.int32)

    def do_row(r, carry):
        row = wid * _RPW + r
        pltpu.sync_copy(x_hbm.at[row], row_v)

        _clear_hist(hist_v)

        def l1(i, c):
            v = row_v[pl.ds(i * _L, _L)]
            ab = lax.bitcast_convert_type(v, jnp.int32) & _ABSM
            b = lax.shift_right_logical(ab, 23)
            plsc.addupdate_scatter(hist_v, [lane_base + b], ones)
            return c

        lax.fori_loop(0, _NSLICES, l1, jnp.int32(0))
        _totals(hist_v, tot_v)
        b1, k2 = _scan_bucket(tot_v, jnp.int32(_K), 256)

        _clear_hist(hist_v)

        def l2(i, w):
            v = row_v[pl.ds(i * _L, _L)]
            ab = lax.bitcast_convert_type(v, jnp.int32) & _ABSM
            hit = lax.shift_right_logical(ab, 23) == b1
            f2 = lax.shift_right_logical(ab, 15) & 0xFF
            plsc.addupdate_scatter(hist_v, [lane_base + f2], ones, mask=hit)
            plsc.store_compressed(s2_v.at[pl.ds(w, _L)], ab, mask=hit)
            return w + jnp.sum(hit.astype(jnp.int32))

        n2 = lax.fori_loop(0, _NSLICES, l2, jnp.int32(0))
        _totals(hist_v, tot_v)
        b2, k3 = _scan_bucket(tot_v, k2, 256)

        _clear_hist(hist_v)

        def l3(i, w):
            off = i * _L
            ab = s2_v[pl.ds(off, _L)]
            valid = (off + iota) < n2
            hit = valid & ((lax.shift_right_logical(ab, 15) & 0xFF) == b2)
            f3 = lax.shift_right_logical(ab, 7) & 0xFF
            plsc.addupdate_scatter(hist_v, [lane_base + f3], ones, mask=hit)
            plsc.store_compressed(s2_v.at[pl.ds(w, _L)], ab, mask=hit)
            return w + jnp.sum(hit.astype(jnp.int32))

        nit2 = (n2 + _L - 1) // _L
        n3 = lax.fori_loop(0, nit2, l3, jnp.int32(0))
        _totals(hist_v, tot_v)
        b3, k4 = _scan_bucket(tot_v, k3, 256)

        _clear_hist(hist_v)

        def l4(i, c):
            off = i * _L
            ab = s2_v[pl.ds(off, _L)]
            valid = (off + iota) < n3
            hit = valid & ((lax.shift_right_logical(ab, 7) & 0xFF) == b3)
            f4 = ab & 0x7F
            plsc.addupdate_scatter(hist_v, [lane_base + f4], ones, mask=hit)
            return c

        nit3 = (n3 + _L - 1) // _L
        lax.fori_loop(0, nit3, l4, jnp.int32(0))
        _totals(hist_v, tot_v)
        b4, _ = _scan_bucket(tot_v, k4, 128)

        t = (
            lax.shift_left(b1, 23)
            | lax.shift_left(b2, 15)
            | lax.shift_left(b3, 7)
            | b4
        )
        plsc.store_scatter(
            thr_v, [jnp.full((_L,), r, jnp.int32)],
            jnp.full((_L,), t, jnp.int32), mask=iota == 0,
        )
        return carry

    lax.fori_loop(0, _RPW, do_row, jnp.int32(0))
    pltpu.sync_copy(thr_v, thr_hbm.at[pl.ds(wid * _RPW, _RPW)])


def _sc_thresholds(x2):
    mesh = plsc.VectorSubcoreMesh(core_axis_name="c", subcore_axis_name="s")
    return pl.kernel(
        _sc_body,
        out_type=jax.ShapeDtypeStruct((_ROWS,), jnp.int32),
        mesh=mesh,
        compiler_params=pltpu.CompilerParams(needs_layout_passes=False),
        scratch_types=[
            pltpu.VMEM((_ROW,), jnp.float32),
            pltpu.VMEM((_ROW + _L,), jnp.int32),
            pltpu.VMEM((256 * _L,), jnp.int32),
            pltpu.VMEM((256 + _L,), jnp.int32),
            pltpu.VMEM((_RPW,), jnp.int32),
        ],
    )(x2)


def _tc_mask_body(x_ref, thr_ref, o_ref):
    xb = x_ref[...]
    ab = lax.bitcast_convert_type(xb, jnp.int32) & _ABSM
    t = thr_ref[:, :1]
    o_ref[...] = jnp.where(ab >= t, xb, jnp.float32(0.0))


def _tc_mask(x2, thrb):
    rpb = 16
    return pl.pallas_call(
        _tc_mask_body,
        grid=(_ROWS // rpb,),
        in_specs=[
            pl.BlockSpec((rpb, _ROW), lambda i: (i, 0)),
            pl.BlockSpec((rpb, 128), lambda i: (i, 0)),
        ],
        out_specs=pl.BlockSpec((rpb, _ROW), lambda i: (i, 0)),
        out_shape=jax.ShapeDtypeStruct((_ROWS, _ROW), jnp.float32),
    )(x2, thrb)


def kernel(x):
    B, C, H, W = x.shape
    x2 = x.reshape(_ROWS, _ROW)
    thr = _sc_thresholds(x2)
    thrb = jnp.broadcast_to(thr[:, None], (_ROWS, 128))
    out = _tc_mask(x2, thrb)
    return out.reshape(B, C, H, W)


# speedup vs baseline: 1.1157x; 1.1157x over previous
import jax
import jax.numpy as jnp
from jax import lax
from jax.experimental import pallas as pl
from jax.experimental.pallas import tpu as pltpu
from jax.experimental.pallas import tpu_sc as plsc

_K = 25088
_ROW = 224 * 224
_ROWS = 768
_NC, _NS, _L = 2, 16, 16
_NW = _NC * _NS
_RPW = _ROWS // _NW
_NSLICES = _ROW // _L
_ABSM = 0x7FFFFFFF


def _clear_hist(hist_v):
    zeros = jnp.zeros((_L,), jnp.int32)

    def body(j, c):
        hist_v[pl.ds(j * _L, _L)] = zeros
        return c

    lax.fori_loop(0, 256 * _L // _L, body, jnp.int32(0))


def _totals(hist_v, tot_v):
    zeros = jnp.zeros((_L,), jnp.int32)

    def body(j, c):
        acc = jnp.zeros((_L,), jnp.int32)
        for l in range(_L):
            acc = acc + hist_v[pl.ds(l * 256 + j * _L, _L)]
            hist_v[pl.ds(l * 256 + j * _L, _L)] = zeros
        tot_v[pl.ds(j * _L, _L)] = acc
        return c

    lax.fori_loop(0, 256 // _L, body, jnp.int32(0))


def _scan_bucket(tot_v, krem, top):
    def cond(c):
        acc, _ = c
        return acc < krem

    def body(c):
        acc, b = c
        b = b - 1
        return acc + tot_v[pl.ds(b, _L)][0], b

    acc, b = lax.while_loop(cond, body, (jnp.int32(0), jnp.int32(top)))
    krem_next = krem - (acc - tot_v[pl.ds(b, _L)][0])
    return b, krem_next


def _sc_body(x_hbm, thr_hbm, row_v, s2_v, hist_v, tot_v, thr_v):
    wid = lax.axis_index("s") * _NC + lax.axis_index("c")
    iota = lax.iota(jnp.int32, _L)
    lane_base = iota * 256
    ones = jnp.ones((_L,), jnp.int32)

    _clear_hist(hist_v)

    def do_row(r, carry):
        row = wid * _RPW + r
        pltpu.sync_copy(x_hbm.at[row], row_v)

        def l1(i, c):
            v = row_v[pl.ds(i * _L, _L)]
            ab = lax.bitcast_convert_type(v, jnp.int32) & _ABSM
            b = lax.shift_right_logical(ab, 23)
            plsc.addupdate_scatter(hist_v, [lane_base + b], ones)
            return c

        lax.fori_loop(0, _NSLICES, l1, jnp.int32(0), unroll=4)
        _totals(hist_v, tot_v)
        b1, k2 = _scan_bucket(tot_v, jnp.int32(_K), 256)

        def l2(i, w):
            v = row_v[pl.ds(i * _L, _L)]
            ab = lax.bitcast_convert_type(v, jnp.int32) & _ABSM
            hit = lax.shift_right_logical(ab, 23) == b1
            f2 = lax.shift_right_logical(ab, 15) & 0xFF
            plsc.addupdate_scatter(hist_v, [lane_base + f2], ones, mask=hit)
            plsc.store_compressed(s2_v.at[pl.ds(w, _L)], ab, mask=hit)
            return w + plsc.all_reduce_population_count(hit)[0]

        n2 = lax.fori_loop(0, _NSLICES, l2, jnp.int32(0), unroll=4)
        _totals(hist_v, tot_v)
        b2, k3 = _scan_bucket(tot_v, k2, 256)

        def l3(i, w):
            off = i * _L
            ab = s2_v[pl.ds(off, _L)]
            valid = (off + iota) < n2
            hit = valid & ((lax.shift_right_logical(ab, 15) & 0xFF) == b2)
            f3 = lax.shift_right_logical(ab, 7) & 0xFF
            plsc.addupdate_scatter(hist_v, [lane_base + f3], ones, mask=hit)
            plsc.store_compressed(s2_v.at[pl.ds(w, _L)], ab, mask=hit)
            return w + plsc.all_reduce_population_count(hit)[0]

        nit2 = (n2 + _L - 1) // _L
        n3 = lax.fori_loop(0, nit2, l3, jnp.int32(0))
        _totals(hist_v, tot_v)
        b3, k4 = _scan_bucket(tot_v, k3, 256)

        def l4(i, c):
            off = i * _L
            ab = s2_v[pl.ds(off, _L)]
            valid = (off + iota) < n3
            hit = valid & ((lax.shift_right_logical(ab, 7) & 0xFF) == b3)
            f4 = ab & 0x7F
            plsc.addupdate_scatter(hist_v, [lane_base + f4], ones, mask=hit)
            return c

        nit3 = (n3 + _L - 1) // _L
        lax.fori_loop(0, nit3, l4, jnp.int32(0))
        _totals(hist_v, tot_v)
        b4, _ = _scan_bucket(tot_v, k4, 128)

        t = (
            lax.shift_left(b1, 23)
            | lax.shift_left(b2, 15)
            | lax.shift_left(b3, 7)
            | b4
        )
        plsc.store_scatter(
            thr_v, [jnp.full((_L,), r, jnp.int32)],
            jnp.full((_L,), t, jnp.int32), mask=iota == 0,
        )
        return carry

    lax.fori_loop(0, _RPW, do_row, jnp.int32(0))
    pltpu.sync_copy(thr_v, thr_hbm.at[pl.ds(wid * _RPW, _RPW)])


def _sc_thresholds(x2):
    mesh = plsc.VectorSubcoreMesh(core_axis_name="c", subcore_axis_name="s")
    return pl.kernel(
        _sc_body,
        out_type=jax.ShapeDtypeStruct((_ROWS,), jnp.int32),
        mesh=mesh,
        compiler_params=pltpu.CompilerParams(needs_layout_passes=False),
        scratch_types=[
            pltpu.VMEM((_ROW,), jnp.float32),
            pltpu.VMEM((_ROW + _L,), jnp.int32),
            pltpu.VMEM((256 * _L,), jnp.int32),
            pltpu.VMEM((256 + _L,), jnp.int32),
            pltpu.VMEM((_RPW,), jnp.int32),
        ],
    )(x2)


def _tc_mask_body(x_ref, thr_ref, o_ref):
    xb = x_ref[...]
    ab = lax.bitcast_convert_type(xb, jnp.int32) & _ABSM
    t = thr_ref[:, :1]
    o_ref[...] = jnp.where(ab >= t, xb, jnp.float32(0.0))


def _tc_mask(x2, thrb):
    rpb = 16
    return pl.pallas_call(
        _tc_mask_body,
        grid=(_ROWS // rpb,),
        in_specs=[
            pl.BlockSpec((rpb, _ROW), lambda i: (i, 0)),
            pl.BlockSpec((rpb, 128), lambda i: (i, 0)),
        ],
        out_specs=pl.BlockSpec((rpb, _ROW), lambda i: (i, 0)),
        out_shape=jax.ShapeDtypeStruct((_ROWS, _ROW), jnp.float32),
    )(x2, thrb)


def kernel(x):
    B, C, H, W = x.shape
    x2 = x.reshape(_ROWS, _ROW)
    thr = _sc_thresholds(x2)
    thrb = jnp.broadcast_to(thr[:, None], (_ROWS, 128))
    out = _tc_mask(x2, thrb)
    return out.reshape(B, C, H, W)


# speedup vs baseline: 1.1618x; 1.0413x over previous
import jax
import jax.numpy as jnp
from jax import lax
from jax.experimental import pallas as pl
from jax.experimental.pallas import tpu as pltpu
from jax.experimental.pallas import tpu_sc as plsc

_K = 25088
_ROW = 224 * 224
_ROWS = 768
_NC, _NS, _L = 2, 16, 16
_NW = _NC * _NS
_RPW = _ROWS // _NW
_NSLICES = _ROW // _L
_HSTRIDE = 257
_ABSM = 0x7FFFFFFF


def _clear_hist(hist_v):
    zeros = jnp.zeros((_L,), jnp.int32)

    def body(j, c):
        hist_v[pl.ds(j * _L, _L)] = zeros
        return c

    lax.fori_loop(0, _HSTRIDE * _L // _L, body, jnp.int32(0))


def _totals(hist_v, tot_v):
    zeros = jnp.zeros((_L,), jnp.int32)

    def body(j, c):
        acc = jnp.zeros((_L,), jnp.int32)
        for l in range(_L):
            acc = acc + hist_v[pl.ds(l * _HSTRIDE + j * _L, _L)]
            hist_v[pl.ds(l * _HSTRIDE + j * _L, _L)] = zeros
        tot_v[pl.ds(j * _L, _L)] = acc
        return c

    lax.fori_loop(0, 256 // _L, body, jnp.int32(0))


def _scan_bucket(tot_v, krem, top):
    def cond(c):
        acc, _ = c
        return acc < krem

    def body(c):
        acc, b = c
        b = b - 1
        return acc + tot_v[pl.ds(b, _L)][0], b

    acc, b = lax.while_loop(cond, body, (jnp.int32(0), jnp.int32(top)))
    krem_next = krem - (acc - tot_v[pl.ds(b, _L)][0])
    return b, krem_next


def _sc_body(x_hbm, thr_hbm, row_v, s2_v, hist_v, tot_v, thr_v):
    wid = lax.axis_index("s") * _NC + lax.axis_index("c")
    iota = lax.iota(jnp.int32, _L)
    lane_base = iota * _HSTRIDE
    ones = jnp.ones((_L,), jnp.int32)

    _clear_hist(hist_v)

    def do_row(r, carry):
        row = wid * _RPW + r
        pltpu.sync_copy(x_hbm.at[row], row_v)

        def l1(i, c):
            v = row_v[pl.ds(i * _L, _L)]
            ab = lax.bitcast_convert_type(v, jnp.int32) & _ABSM
            b = lax.shift_right_logical(ab, 23)
            plsc.addupdate_scatter(hist_v, [lane_base + b], ones)
            return c

        lax.fori_loop(0, _NSLICES, l1, jnp.int32(0), unroll=4)
        _totals(hist_v, tot_v)
        b1, k2 = _scan_bucket(tot_v, jnp.int32(_K), 256)

        def l2(i, w):
            v = row_v[pl.ds(i * _L, _L)]
            ab = lax.bitcast_convert_type(v, jnp.int32) & _ABSM
            hit = lax.shift_right_logical(ab, 23) == b1
            f2 = lax.shift_right_logical(ab, 15) & 0xFF
            plsc.addupdate_scatter(hist_v, [lane_base + f2], ones, mask=hit)
            plsc.store_compressed(s2_v.at[pl.ds(w, _L)], ab, mask=hit)
            return w + plsc.all_reduce_population_count(hit)[0]

        n2 = lax.fori_loop(0, _NSLICES, l2, jnp.int32(0), unroll=4)
        _totals(hist_v, tot_v)
        b2, k3 = _scan_bucket(tot_v, k2, 256)

        def l3(i, w):
            off = i * _L
            ab = s2_v[pl.ds(off, _L)]
            valid = (off + iota) < n2
            hit = valid & ((lax.shift_right_logical(ab, 15) & 0xFF) == b2)
            f3 = lax.shift_right_logical(ab, 7) & 0xFF
            plsc.addupdate_scatter(hist_v, [lane_base + f3], ones, mask=hit)
            plsc.store_compressed(s2_v.at[pl.ds(w, _L)], ab, mask=hit)
            return w + plsc.all_reduce_population_count(hit)[0]

        nit2 = (n2 + _L - 1) // _L
        n3 = lax.fori_loop(0, nit2, l3, jnp.int32(0))
        _totals(hist_v, tot_v)
        b3, k4 = _scan_bucket(tot_v, k3, 256)

        def l4(i, c):
            off = i * _L
            ab = s2_v[pl.ds(off, _L)]
            valid = (off + iota) < n3
            hit = valid & ((lax.shift_right_logical(ab, 7) & 0xFF) == b3)
            f4 = ab & 0x7F
            plsc.addupdate_scatter(hist_v, [lane_base + f4], ones, mask=hit)
            return c

        nit3 = (n3 + _L - 1) // _L
        lax.fori_loop(0, nit3, l4, jnp.int32(0))
        _totals(hist_v, tot_v)
        b4, _ = _scan_bucket(tot_v, k4, 128)

        t = (
            lax.shift_left(b1, 23)
            | lax.shift_left(b2, 15)
            | lax.shift_left(b3, 7)
            | b4
        )
        plsc.store_scatter(
            thr_v, [jnp.full((_L,), r, jnp.int32)],
            jnp.full((_L,), t, jnp.int32), mask=iota == 0,
        )
        return carry

    lax.fori_loop(0, _RPW, do_row, jnp.int32(0))
    pltpu.sync_copy(thr_v, thr_hbm.at[pl.ds(wid * _RPW, _RPW)])


def _sc_thresholds(x2):
    mesh = plsc.VectorSubcoreMesh(core_axis_name="c", subcore_axis_name="s")
    return pl.kernel(
        _sc_body,
        out_type=jax.ShapeDtypeStruct((_ROWS,), jnp.int32),
        mesh=mesh,
        compiler_params=pltpu.CompilerParams(needs_layout_passes=False),
        scratch_types=[
            pltpu.VMEM((_ROW,), jnp.float32),
            pltpu.VMEM((_ROW + _L,), jnp.int32),
            pltpu.VMEM((_HSTRIDE * _L,), jnp.int32),
            pltpu.VMEM((256 + _L,), jnp.int32),
            pltpu.VMEM((_RPW,), jnp.int32),
        ],
    )(x2)


def _tc_mask_body(x_ref, thr_ref, o_ref):
    xb = x_ref[...]
    ab = lax.bitcast_convert_type(xb, jnp.int32) & _ABSM
    t = thr_ref[:, :1]
    o_ref[...] = jnp.where(ab >= t, xb, jnp.float32(0.0))


def _tc_mask(x2, thrb):
    rpb = 16
    return pl.pallas_call(
        _tc_mask_body,
        grid=(_ROWS // rpb,),
        in_specs=[
            pl.BlockSpec((rpb, _ROW), lambda i: (i, 0)),
            pl.BlockSpec((rpb, 128), lambda i: (i, 0)),
        ],
        out_specs=pl.BlockSpec((rpb, _ROW), lambda i: (i, 0)),
        out_shape=jax.ShapeDtypeStruct((_ROWS, _ROW), jnp.float32),
    )(x2, thrb)


def kernel(x):
    B, C, H, W = x.shape
    x2 = x.reshape(_ROWS, _ROW)
    thr = _sc_thresholds(x2)
    thrb = jnp.broadcast_to(thr[:, None], (_ROWS, 128))
    out = _tc_mask(x2, thrb)
    return out.reshape(B, C, H, W)


# speedup vs baseline: 1.9403x; 1.6701x over previous
import jax
import jax.numpy as jnp
from jax import lax
from jax.experimental import pallas as pl
from jax.experimental.pallas import tpu as pltpu
from jax.experimental.pallas import tpu_sc as plsc

_K = 25088
_ROW = 224 * 224
_ROWS = 768
_NC, _NS, _L = 2, 16, 16
_NW = _NC * _NS
_RPW = _ROWS // _NW
_NSLICES = _ROW // _L
_HSTRIDE = 257
_ABSM = 0x7FFFFFFF


def _clear_hist(hist_v):
    zeros = jnp.zeros((_L,), jnp.int32)

    def body(j, c):
        hist_v[pl.ds(j * _L, _L)] = zeros
        return c

    lax.fori_loop(0, _HSTRIDE * _L // _L, body, jnp.int32(0))


def _totals(hist_v, tot_v):
    zeros = jnp.zeros((_L,), jnp.int32)

    def body(j, c):
        vals = [hist_v[pl.ds(l * _HSTRIDE + j * _L, _L)] for l in range(_L)]
        for l in range(_L):
            hist_v[pl.ds(l * _HSTRIDE + j * _L, _L)] = zeros
        while len(vals) > 1:
            vals = [
                vals[m] + vals[m + 1] if m + 1 < len(vals) else vals[m]
                for m in range(0, len(vals), 2)
            ]
        tot_v[pl.ds(j * _L, _L)] = vals[0]
        return c

    lax.fori_loop(0, 256 // _L, body, jnp.int32(0))


def _scan_bucket(tot_v, krem, top):
    def cond(c):
        acc, _ = c
        return acc < krem

    def body(c):
        acc, b = c
        b = b - 1
        return acc + tot_v[pl.ds(b, _L)][0], b

    acc, b = lax.while_loop(cond, body, (jnp.int32(0), jnp.int32(top)))
    krem_next = krem - (acc - tot_v[pl.ds(b, _L)][0])
    return b, krem_next


def _sc_body(x_hbm, thr_hbm, row_v, s2_v, hist_v, tot_v, thr_v):
    wid = lax.axis_index("s") * _NC + lax.axis_index("c")
    iota = lax.iota(jnp.int32, _L)
    lane_base = iota * _HSTRIDE
    ones = jnp.ones((_L,), jnp.int32)

    _clear_hist(hist_v)

    def do_row(r, carry):
        row = wid * _RPW + r
        pltpu.sync_copy(x_hbm.at[row], row_v)

        g1 = 8

        def l1(i, c):
            base = i * (_L * g1)
            vs = [row_v[pl.ds(base + u * _L, _L)] for u in range(g1)]
            bs = [
                lax.shift_right_logical(
                    lax.bitcast_convert_type(v, jnp.int32) & _ABSM, 23
                )
                for v in vs
            ]
            for b in bs:
                plsc.addupdate_scatter(hist_v, [lane_base + b], ones)
            return c

        lax.fori_loop(0, _NSLICES // g1, l1, jnp.int32(0))
        _totals(hist_v, tot_v)
        b1, k2 = _scan_bucket(tot_v, jnp.int32(_K), 256)

        g2 = 4

        def l2(i, w):
            base = i * (_L * g2)
            abs_ = [
                lax.bitcast_convert_type(
                    row_v[pl.ds(base + u * _L, _L)], jnp.int32
                )
                & _ABSM
                for u in range(g2)
            ]
            hits = [lax.shift_right_logical(ab, 23) == b1 for ab in abs_]
            f2s = [lax.shift_right_logical(ab, 15) & 0xFF for ab in abs_]
            pcs = [plsc.all_reduce_population_count(h)[0] for h in hits]
            for ab, hit, f2 in zip(abs_, hits, f2s):
                plsc.addupdate_scatter(hist_v, [lane_base + f2], ones, mask=hit)
                plsc.store_compressed(s2_v.at[pl.ds(w, _L)], ab, mask=hit)
                w = w + pcs.pop(0)
            return w

        n2 = lax.fori_loop(0, _NSLICES // g2, l2, jnp.int32(0))
        _totals(hist_v, tot_v)
        b2, k3 = _scan_bucket(tot_v, k2, 256)

        g3 = 4

        def l3(i, w):
            base = i * (_L * g3)
            abs_ = [s2_v[pl.ds(base + u * _L, _L)] for u in range(g3)]
            hits = [
                ((base + u * _L + iota) < n2)
                & ((lax.shift_right_logical(ab, 15) & 0xFF) == b2)
                for u, ab in enumerate(abs_)
            ]
            f3s = [lax.shift_right_logical(ab, 7) & 0xFF for ab in abs_]
            pcs = [plsc.all_reduce_population_count(h)[0] for h in hits]
            for ab, hit, f3 in zip(abs_, hits, f3s):
                plsc.addupdate_scatter(hist_v, [lane_base + f3], ones, mask=hit)
                plsc.store_compressed(s2_v.at[pl.ds(w, _L)], ab, mask=hit)
                w = w + pcs.pop(0)
            return w

        nit2 = (n2 + _L * g3 - 1) // (_L * g3)
        n3 = lax.fori_loop(0, nit2, l3, jnp.int32(0))
        _totals(hist_v, tot_v)
        b3, k4 = _scan_bucket(tot_v, k3, 256)

        def l4(i, c):
            off = i * _L
            ab = s2_v[pl.ds(off, _L)]
            valid = (off + iota) < n3
            hit = valid & ((lax.shift_right_logical(ab, 7) & 0xFF) == b3)
            f4 = ab & 0x7F
            plsc.addupdate_scatter(hist_v, [lane_base + f4], ones, mask=hit)
            return c

        nit3 = (n3 + _L - 1) // _L
        lax.fori_loop(0, nit3, l4, jnp.int32(0))
        _totals(hist_v, tot_v)
        b4, _ = _scan_bucket(tot_v, k4, 128)

        t = (
            lax.shift_left(b1, 23)
            | lax.shift_left(b2, 15)
            | lax.shift_left(b3, 7)
            | b4
        )
        plsc.store_scatter(
            thr_v, [jnp.full((_L,), r, jnp.int32)],
            jnp.full((_L,), t, jnp.int32), mask=iota == 0,
        )
        return carry

    lax.fori_loop(0, _RPW, do_row, jnp.int32(0))
    pltpu.sync_copy(thr_v, thr_hbm.at[pl.ds(wid * _RPW, _RPW)])


def _sc_thresholds(x2):
    mesh = plsc.VectorSubcoreMesh(core_axis_name="c", subcore_axis_name="s")
    return pl.kernel(
        _sc_body,
        out_type=jax.ShapeDtypeStruct((_ROWS,), jnp.int32),
        mesh=mesh,
        compiler_params=pltpu.CompilerParams(needs_layout_passes=False),
        scratch_types=[
            pltpu.VMEM((_ROW,), jnp.float32),
            pltpu.VMEM((_ROW + _L,), jnp.int32),
            pltpu.VMEM((_HSTRIDE * _L,), jnp.int32),
            pltpu.VMEM((256 + _L,), jnp.int32),
            pltpu.VMEM((_RPW,), jnp.int32),
        ],
    )(x2)


def _tc_mask_body(x_ref, thr_ref, o_ref):
    xb = x_ref[...]
    ab = lax.bitcast_convert_type(xb, jnp.int32) & _ABSM
    t = thr_ref[:, :1]
    o_ref[...] = jnp.where(ab >= t, xb, jnp.float32(0.0))


def _tc_mask(x2, thrb):
    rpb = 16
    return pl.pallas_call(
        _tc_mask_body,
        grid=(_ROWS // rpb,),
        in_specs=[
            pl.BlockSpec((rpb, _ROW), lambda i: (i, 0)),
            pl.BlockSpec((rpb, 128), lambda i: (i, 0)),
        ],
        out_specs=pl.BlockSpec((rpb, _ROW), lambda i: (i, 0)),
        out_shape=jax.ShapeDtypeStruct((_ROWS, _ROW), jnp.float32),
    )(x2, thrb)


def kernel(x):
    B, C, H, W = x.shape
    x2 = x.reshape(_ROWS, _ROW)
    thr = _sc_thresholds(x2)
    thrb = jnp.broadcast_to(thr[:, None], (_ROWS, 128))
    out = _tc_mask(x2, thrb)
    return out.reshape(B, C, H, W)


# speedup vs baseline: 2.1670x; 1.1168x over previous
import jax
import jax.numpy as jnp
from jax import lax
from jax.experimental import pallas as pl
from jax.experimental.pallas import tpu as pltpu
from jax.experimental.pallas import tpu_sc as plsc

_K = 25088
_ROW = 224 * 224
_ROWS = 768
_NC, _NS, _L = 2, 16, 16
_NW = _NC * _NS
_RPW = _ROWS // _NW
_NSLICES = _ROW // _L
_HSTRIDE = 257
_ABSM = 0x7FFFFFFF
_DELTA = 600000


def _clear_hist(hist_v):
    zeros = jnp.zeros((_L,), jnp.int32)

    def body(j, c):
        hist_v[pl.ds(j * _L, _L)] = zeros
        return c

    lax.fori_loop(0, _HSTRIDE * _L // _L, body, jnp.int32(0))


def _totals(hist_v, tot_v):
    zeros = jnp.zeros((_L,), jnp.int32)

    def body(j, c):
        vals = [hist_v[pl.ds(l * _HSTRIDE + j * _L, _L)] for l in range(_L)]
        for l in range(_L):
            hist_v[pl.ds(l * _HSTRIDE + j * _L, _L)] = zeros
        while len(vals) > 1:
            vals = [
                vals[m] + vals[m + 1] if m + 1 < len(vals) else vals[m]
                for m in range(0, len(vals), 2)
            ]
        tot_v[pl.ds(j * _L, _L)] = vals[0]
        return c

    lax.fori_loop(0, 256 // _L, body, jnp.int32(0))


def _scan_bucket(tot_v, krem, top):
    def cond(c):
        acc, _ = c
        return acc < krem

    def body(c):
        acc, b = c
        b = b - 1
        return acc + tot_v[pl.ds(b, _L)][0], b

    acc, b = lax.while_loop(cond, body, (jnp.int32(0), jnp.int32(top)))
    krem_next = krem - (acc - tot_v[pl.ds(b, _L)][0])
    return b, krem_next


def _sc_body(x_hbm, thr_hbm, row_v, s2_v, hist_v, tot_v, thr_v):
    wid = lax.axis_index("s") * _NC + lax.axis_index("c")
    iota = lax.iota(jnp.int32, _L)
    lane_base = iota * _HSTRIDE
    ones = jnp.ones((_L,), jnp.int32)

    _clear_hist(hist_v)

    def surv_pass(n, sel_shift, sel_val, hist_shift, hmask, compact):
        g = 4

        def body(i, w):
            base = i * (_L * g)
            abs_ = [s2_v[pl.ds(base + u * _L, _L)] for u in range(g)]
            valids = [(base + u * _L + iota) < n for u in range(g)]
            if sel_shift is None:
                hits = valids
            else:
                hits = [
                    v
                    & (
                        (lax.shift_right_logical(ab, sel_shift) & 0xFF)
                        == sel_val
                    )
                    for v, ab in zip(valids, abs_)
                ]
            fs = [
                lax.shift_right_logical(ab, hist_shift) & hmask
                for ab in abs_
            ]
            if compact:
                pcs = [plsc.all_reduce_population_count(h)[0] for h in hits]
            for u, (ab, hit, f) in enumerate(zip(abs_, hits, fs)):
                plsc.addupdate_scatter(hist_v, [lane_base + f], ones, mask=hit)
                if compact:
                    plsc.store_compressed(s2_v.at[pl.ds(w, _L)], ab, mask=hit)
                    w = w + pcs[u]
            return w

        nit = (n + _L * g - 1) // (_L * g)
        return lax.fori_loop(0, nit, body, jnp.int32(0))

    def select_surv(n, krem):
        surv_pass(n, None, None, 23, 0xFF, False)
        _totals(hist_v, tot_v)
        c1, krem = _scan_bucket(tot_v, krem, 256)

        n = surv_pass(n, 23, c1, 15, 0xFF, True)
        _totals(hist_v, tot_v)
        c2, krem = _scan_bucket(tot_v, krem, 256)

        n = surv_pass(n, 15, c2, 7, 0xFF, True)
        _totals(hist_v, tot_v)
        c3, krem = _scan_bucket(tot_v, krem, 256)

        surv_pass(n, 7, c3, 0, 0x7F, False)
        _totals(hist_v, tot_v)
        c4, _ = _scan_bucket(tot_v, krem, 128)

        return (
            lax.shift_left(c1, 23)
            | lax.shift_left(c2, 15)
            | lax.shift_left(c3, 7)
            | c4
        )

    def do_row(r, t_prev):
        row = wid * _RPW + r
        pltpu.sync_copy(x_hbm.at[row], row_v)

        lo = jnp.maximum(t_prev - _DELTA, 0)
        hi = t_prev + _DELTA

        g = 4

        def win(i, carry):
            w, c = carry
            base = i * (_L * g)
            abs_ = [
                lax.bitcast_convert_type(
                    row_v[pl.ds(base + u * _L, _L)], jnp.int32
                )
                & _ABSM
                for u in range(g)
            ]
            mhis = [ab > hi for ab in abs_]
            mins = [
                (ab >= lo) & jnp.logical_not(mh)
                for ab, mh in zip(abs_, mhis)
            ]
            pcs = [plsc.all_reduce_population_count(m)[0] for m in mins]
            for u, (ab, mh, mi) in enumerate(zip(abs_, mhis, mins)):
                c = c + mh.astype(jnp.int32)
                plsc.store_compressed(s2_v.at[pl.ds(w, _L)], ab, mask=mi)
                w = w + pcs[u]
            return w, c

        n_win, cvec = lax.fori_loop(
            0,
            _NSLICES // g,
            win,
            (jnp.int32(0), jnp.zeros((_L,), jnp.int32)),
        )
        cnt_hi = jnp.sum(cvec)
        ok = (cnt_hi < _K) & (cnt_hi + n_win >= _K)

        @pl.when(jnp.logical_not(ok))
        def _():
            def cp(i, c):
                base = i * (_L * 4)
                for u in range(4):
                    s2_v[pl.ds(base + u * _L, _L)] = (
                        lax.bitcast_convert_type(
                            row_v[pl.ds(base + u * _L, _L)], jnp.int32
                        )
                        & _ABSM
                    )
                return c

            lax.fori_loop(0, _NSLICES // 4, cp, jnp.int32(0))

        n_eff = jnp.where(ok, n_win, _ROW)
        k_eff = jnp.where(ok, _K - cnt_hi, _K)
        t = select_surv(n_eff, k_eff)

        plsc.store_scatter(
            thr_v, [jnp.full((_L,), r, jnp.int32)],
            jnp.full((_L,), t, jnp.int32), mask=iota == 0,
        )
        return t

    lax.fori_loop(0, _RPW, do_row, jnp.int32(0))
    pltpu.sync_copy(thr_v, thr_hbm.at[pl.ds(wid * _RPW, _RPW)])


def _sc_thresholds(x2):
    mesh = plsc.VectorSubcoreMesh(core_axis_name="c", subcore_axis_name="s")
    return pl.kernel(
        _sc_body,
        out_type=jax.ShapeDtypeStruct((_ROWS,), jnp.int32),
        mesh=mesh,
        compiler_params=pltpu.CompilerParams(needs_layout_passes=False),
        scratch_types=[
            pltpu.VMEM((_ROW,), jnp.float32),
            pltpu.VMEM((_ROW + _L,), jnp.int32),
            pltpu.VMEM((_HSTRIDE * _L,), jnp.int32),
            pltpu.VMEM((256 + _L,), jnp.int32),
            pltpu.VMEM((_RPW,), jnp.int32),
        ],
    )(x2)


def _tc_mask_body(x_ref, thr_ref, o_ref):
    xb = x_ref[...]
    ab = lax.bitcast_convert_type(xb, jnp.int32) & _ABSM
    t = thr_ref[:, :1]
    o_ref[...] = jnp.where(ab >= t, xb, jnp.float32(0.0))


def _tc_mask(x2, thrb):
    rpb = 16
    return pl.pallas_call(
        _tc_mask_body,
        grid=(_ROWS // rpb,),
        in_specs=[
            pl.BlockSpec((rpb, _ROW), lambda i: (i, 0)),
            pl.BlockSpec((rpb, 128), lambda i: (i, 0)),
        ],
        out_specs=pl.BlockSpec((rpb, _ROW), lambda i: (i, 0)),
        out_shape=jax.ShapeDtypeStruct((_ROWS, _ROW), jnp.float32),
    )(x2, thrb)


def kernel(x):
    B, C, H, W = x.shape
    x2 = x.reshape(_ROWS, _ROW)
    thr = _sc_thresholds(x2)
    thrb = jnp.broadcast_to(thr[:, None], (_ROWS, 128))
    out = _tc_mask(x2, thrb)
    return out.reshape(B, C, H, W)


# speedup vs baseline: 4.1817x; 1.9297x over previous
import jax
import jax.numpy as jnp
from jax import lax
from jax.experimental import pallas as pl
from jax.experimental.pallas import tpu as pltpu
from jax.experimental.pallas import tpu_sc as plsc

_K = 25088
_ROW = 224 * 224
_ROWS = 768
_NC, _NS, _L = 2, 16, 16
_NW = _NC * _NS
_RPW = _ROWS // _NW
_NSLICES = _ROW // _L
_HSTRIDE = 257
_ABSM = 0x7FFFFFFF
_DELTA = 600000


def _clear_hist(hist_v):
    zeros = jnp.zeros((_L,), jnp.int32)

    def body(j, c):
        hist_v[pl.ds(j * _L, _L)] = zeros
        return c

    lax.fori_loop(0, _HSTRIDE * _L // _L, body, jnp.int32(0))


def _totals(hist_v, tot_v):
    zeros = jnp.zeros((_L,), jnp.int32)

    def body(j, c):
        vals = [hist_v[pl.ds(l * _HSTRIDE + j * _L, _L)] for l in range(_L)]
        for l in range(_L):
            hist_v[pl.ds(l * _HSTRIDE + j * _L, _L)] = zeros
        while len(vals) > 1:
            vals = [
                vals[m] + vals[m + 1] if m + 1 < len(vals) else vals[m]
                for m in range(0, len(vals), 2)
            ]
        tot_v[pl.ds(j * _L, _L)] = vals[0]
        return c

    lax.fori_loop(0, 256 // _L, body, jnp.int32(0))


def _scan_bucket(tot_v, krem, top):
    iota = lax.iota(jnp.int32, _L)

    def cond(c):
        acc, _ = c
        return acc < krem

    def body(c):
        acc, j = c
        j = j - 1
        return acc + jnp.sum(tot_v[pl.ds(j * _L, _L)]), j

    acc, j = lax.while_loop(cond, body, (jnp.int32(0), jnp.int32(top // _L)))
    t = tot_v[pl.ds(j * _L, _L)]
    above_chunk = acc - jnp.sum(t)
    suff = above_chunk + lax.rev(plsc.cumsum(lax.rev(t, (0,))), (0,))
    m = suff >= krem
    istar = plsc.all_reduce_population_count(m)[0] - 1
    b = j * _L + istar
    count_above = jnp.max(jnp.where(iota == istar, suff - t, 0))
    return b, krem - count_above


def _sc_body(x_hbm, thr_hbm, row_v, s2_v, hist_v, tot_v, thr_v, dma_sem):
    wid = lax.axis_index("s") * _NC + lax.axis_index("c")
    iota = lax.iota(jnp.int32, _L)
    lane_base = iota * _HSTRIDE
    ones = jnp.ones((_L,), jnp.int32)

    pltpu.async_copy(x_hbm.at[wid * _RPW], row_v, dma_sem)
    _clear_hist(hist_v)

    def surv_pass(n, sel_shift, sel_val, hist_shift, hmask, compact):
        g = 4

        def body(i, w):
            base = i * (_L * g)
            abs_ = [s2_v[pl.ds(base + u * _L, _L)] for u in range(g)]
            valids = [(base + u * _L + iota) < n for u in range(g)]
            if sel_shift is None:
                hits = valids
            else:
                hits = [
                    v
                    & (
                        (lax.shift_right_logical(ab, sel_shift) & 0xFF)
                        == sel_val
                    )
                    for v, ab in zip(valids, abs_)
                ]
            fs = [
                lax.shift_right_logical(ab, hist_shift) & hmask
                for ab in abs_
            ]
            if compact:
                pcs = [plsc.all_reduce_population_count(h)[0] for h in hits]
            for u, (ab, hit, f) in enumerate(zip(abs_, hits, fs)):
                plsc.addupdate_scatter(hist_v, [lane_base + f], ones, mask=hit)
                if compact:
                    plsc.store_compressed(s2_v.at[pl.ds(w, _L)], ab, mask=hit)
                    w = w + pcs[u]
            return w

        nit = (n + _L * g - 1) // (_L * g)
        return lax.fori_loop(0, nit, body, jnp.int32(0))

    def select_surv(n, krem):
        surv_pass(n, None, None, 23, 0xFF, False)
        _totals(hist_v, tot_v)
        c1, krem = _scan_bucket(tot_v, krem, 256)

        n = surv_pass(n, 23, c1, 15, 0xFF, True)
        _totals(hist_v, tot_v)
        c2, krem = _scan_bucket(tot_v, krem, 256)

        n = surv_pass(n, 15, c2, 7, 0xFF, True)
        _totals(hist_v, tot_v)
        c3, krem = _scan_bucket(tot_v, krem, 256)

        surv_pass(n, 7, c3, 0, 0x7F, False)
        _totals(hist_v, tot_v)
        c4, _ = _scan_bucket(tot_v, krem, 128)

        return (
            lax.shift_left(c1, 23)
            | lax.shift_left(c2, 15)
            | lax.shift_left(c3, 7)
            | c4
        )

    def do_row(r, t_prev):
        row = wid * _RPW + r
        pltpu.make_async_copy(x_hbm.at[row], row_v, dma_sem).wait()

        lo = jnp.maximum(t_prev - _DELTA, 0)
        hi = t_prev + _DELTA
        span = lax.bitcast_convert_type(hi - lo, jnp.uint32)

        g = 14

        def win(i, carry):
            w, c0, c1, c2, c3 = carry
            cs = [c0, c1, c2, c3]
            base = i * (_L * g)
            abs_ = [
                lax.bitcast_convert_type(
                    row_v[pl.ds(base + u * _L, _L)], jnp.int32
                )
                & _ABSM
                for u in range(g)
            ]
            mhis = [ab > hi for ab in abs_]
            mins = [
                lax.bitcast_convert_type(ab - lo, jnp.uint32) <= span
                for ab in abs_
            ]
            pcs = [plsc.all_reduce_population_count(m)[0] for m in mins]
            for u, (ab, mh, mi) in enumerate(zip(abs_, mhis, mins)):
                cs[u % 4] = cs[u % 4] + mh.astype(jnp.int32)
                plsc.store_compressed(s2_v.at[pl.ds(w, _L)], ab, mask=mi)
                w = w + pcs[u]
            return w, cs[0], cs[1], cs[2], cs[3]

        cz = jnp.zeros((_L,), jnp.int32)
        n_win, c0, c1, c2, c3 = lax.fori_loop(
            0, _NSLICES // g, win, (jnp.int32(0), cz, cz, cz, cz)
        )
        cnt_hi = jnp.sum((c0 + c1) + (c2 + c3))
        ok = (cnt_hi < _K) & (cnt_hi + n_win >= _K)

        @pl.when(jnp.logical_not(ok))
        def _():
            def cp(i, c):
                base = i * (_L * 4)
                for u in range(4):
                    s2_v[pl.ds(base + u * _L, _L)] = (
                        lax.bitcast_convert_type(
                            row_v[pl.ds(base + u * _L, _L)], jnp.int32
                        )
                        & _ABSM
                    )
                return c

            lax.fori_loop(0, _NSLICES // 4, cp, jnp.int32(0))

        @pl.when(r + 1 < _RPW)
        def _():
            pltpu.async_copy(x_hbm.at[row + 1], row_v, dma_sem)

        n_eff = jnp.where(ok, n_win, _ROW)
        k_eff = jnp.where(ok, _K - cnt_hi, _K)
        t = select_surv(n_eff, k_eff)

        plsc.store_scatter(
            thr_v, [jnp.full((_L,), r, jnp.int32)],
            jnp.full((_L,), t, jnp.int32), mask=iota == 0,
        )
        return t

    lax.fori_loop(0, _RPW, do_row, jnp.int32(0))
    pltpu.sync_copy(thr_v, thr_hbm.at[pl.ds(wid * _RPW, _RPW)])


def _sc_thresholds(x2):
    mesh = plsc.VectorSubcoreMesh(core_axis_name="c", subcore_axis_name="s")
    return pl.kernel(
        _sc_body,
        out_type=jax.ShapeDtypeStruct((_ROWS,), jnp.int32),
        mesh=mesh,
        compiler_params=pltpu.CompilerParams(needs_layout_passes=False),
        scratch_types=[
            pltpu.VMEM((_ROW,), jnp.float32),
            pltpu.VMEM((_ROW + _L,), jnp.int32),
            pltpu.VMEM((_HSTRIDE * _L,), jnp.int32),
            pltpu.VMEM((256 + _L,), jnp.int32),
            pltpu.VMEM((_RPW,), jnp.int32),
            pltpu.SemaphoreType.DMA,
        ],
    )(x2)


_CB = 8


def _tc_mask_body(x_ref, thr_ref, o_ref):
    xb = x_ref[...]
    ab = lax.bitcast_convert_type(xb, jnp.int32) & _ABSM
    t = thr_ref[:, :1].reshape(1, _CB, 1, 1)
    o_ref[...] = jnp.where(ab >= t, xb, jnp.float32(0.0))


def _tc_mask(x, thrb):
    B, C, H, W = x.shape
    cpb = C // _CB

    return pl.pallas_call(
        _tc_mask_body,
        grid=(B * cpb,),
        in_specs=[
            pl.BlockSpec(
                (1, _CB, H, W), lambda i: (i // cpb, i % cpb, 0, 0)
            ),
            pl.BlockSpec((_CB, 128), lambda i: (i, 0)),
        ],
        out_specs=pl.BlockSpec(
            (1, _CB, H, W), lambda i: (i // cpb, i % cpb, 0, 0)
        ),
        out_shape=jax.ShapeDtypeStruct((B, C, H, W), jnp.float32),
    )(x, thrb)


def kernel(x):
    B, C, H, W = x.shape
    x2 = x.reshape(_ROWS, _ROW)
    thr = _sc_thresholds(x2)
    thrb = jnp.broadcast_to(thr[:, None], (_ROWS, 128))
    return _tc_mask(x, thrb)
